# baseline (speedup 1.0000x reference)
import functools

import jax
import jax.numpy as jnp
from jax import lax
from jax.experimental import pallas as pl
from jax.experimental.pallas import tpu as pltpu
from jax.experimental.pallas import tpu_sc as plsc

BATCH = 16384
ACTIONS = 16

_info = plsc.get_sparse_core_info()
_NC, _NS, _L = _info.num_cores, _info.num_subcores, _info.num_lanes
_NW = _NC * _NS
_B_PER_W = BATCH // _NW
_NGROUP = _B_PER_W // _L


def _sc_body(x_hbm, tab_hbm, out_hbm, idx_flat, blk0, blk1, blk2, outbuf,
             sem0, sem1, sem2):
    wid = lax.axis_index("s") * _NC + lax.axis_index("c")
    base = wid * _B_PER_W

    pltpu.sync_copy(x_hbm.at[pl.ds(base, _B_PER_W)], idx_flat)

    lane = lax.iota(jnp.int32, _L)

    def fire(g, blocks, sem):
        iv = idx_flat[pl.ds(g * _L, _L)]
        blk = iv >> 7
        for k in range(_L):
            off = blk[k] * 128
            pltpu.async_copy(
                tab_hbm.at[pl.ds(0, 8), pl.ds(off, 128)],
                blocks.at[pl.ds(0, 8), pl.ds(k * 128, 128)],
                sem,
            )
            pltpu.async_copy(
                tab_hbm.at[pl.ds(8, 8), pl.ds(off, 128)],
                blocks.at[pl.ds(8, 8), pl.ds(k * 128, 128)],
                sem,
            )

    def compute(g, blocks, sem):
        pltpu.make_async_copy(
            tab_hbm.at[:, pl.ds(0, _L * 128)], blocks, sem
        ).wait()
        iv = idx_flat[pl.ds(g * _L, _L)]
        col = lane * 128 + (iv & 127)
        vs = [
            plsc.load_gather(blocks, [jnp.full((_L,), j, jnp.int32), col])
            for j in range(ACTIONS)
        ]
        m = vs[0]
        for j in range(1, ACTIONS):
            m = jnp.maximum(m, vs[j])
        es = [jnp.exp(v - m) for v in vs]
        s = es[0]
        for j in range(1, ACTIONS):
            s = s + es[j]
        r = 1.0 / s
        opos = g * _L + lane
        for j in range(ACTIONS):
            plsc.store_scatter(
                outbuf, [jnp.full((_L,), j, jnp.int32), opos], es[j] * r
            )

    bufs = ((blk0, sem0), (blk1, sem1), (blk2, sem2))

    fire(0, blk0, sem0)
    fire(1, blk1, sem1)

    def triple(p, carry):
        g = p * 3
        for q in range(3):
            compute(g + q, *bufs[q])
            fire(g + q + 2, *bufs[(q + 2) % 3])
        return carry

    lax.fori_loop(0, _NGROUP // 3, triple, 0)
    compute(30, blk0, sem0)
    compute(31, blk1, sem1)

    pltpu.sync_copy(outbuf, out_hbm.at[:, pl.ds(base, _B_PER_W)])


@jax.jit
def _run(x, table):
    tab_t = table.T
    mesh = plsc.VectorSubcoreMesh(core_axis_name="c", subcore_axis_name="s")
    kern = functools.partial(
        pl.kernel,
        out_type=jax.ShapeDtypeStruct((ACTIONS, BATCH), jnp.float32),
        mesh=mesh,
        scratch_types=[
            pltpu.VMEM((_B_PER_W,), jnp.int32),
            pltpu.VMEM((ACTIONS, _L * 128), jnp.float32),
            pltpu.VMEM((ACTIONS, _L * 128), jnp.float32),
            pltpu.VMEM((ACTIONS, _L * 128), jnp.float32),
            pltpu.VMEM((ACTIONS, _B_PER_W), jnp.float32),
            pltpu.SemaphoreType.DMA,
            pltpu.SemaphoreType.DMA,
            pltpu.SemaphoreType.DMA,
        ],
        compiler_params=pltpu.CompilerParams(needs_layout_passes=False),
    )(_sc_body)
    out = kern(x.astype(jnp.int32), tab_t)
    return out.T


def kernel(x, table):
    return _run(x, table)


# speedup vs baseline: 1.0029x; 1.0029x over previous
import functools

import jax
import jax.numpy as jnp
from jax import lax
from jax.experimental import pallas as pl
from jax.experimental.pallas import tpu as pltpu
from jax.experimental.pallas import tpu_sc as plsc

BATCH = 16384
ACTIONS = 16

_info = plsc.get_sparse_core_info()
_NC, _NS, _L = _info.num_cores, _info.num_subcores, _info.num_lanes
_NW = _NC * _NS
_B_PER_W = BATCH // _NW
_NGROUP = _B_PER_W // _L


def _sc_body(x_hbm, tab_hbm, out_hbm, idx_flat, blk0, blk1, blk2, outbuf,
             sem0, sem1, sem2):
    wid = lax.axis_index("s") * _NC + lax.axis_index("c")
    base = wid * _B_PER_W

    pltpu.sync_copy(x_hbm.at[pl.ds(base, _B_PER_W)], idx_flat)

    lane = lax.iota(jnp.int32, _L)

    def fire(g, blocks, sem):
        iv = idx_flat[pl.ds(g * _L, _L)]
        blk = iv >> 7
        for k in range(_L):
            pltpu.async_copy(
                tab_hbm.at[:, pl.ds(blk[k] * 128, 128)],
                blocks.at[:, pl.ds(k * 128, 128)],
                sem,
            )

    def compute(g, blocks, sem):
        pltpu.make_async_copy(
            tab_hbm.at[:, pl.ds(0, _L * 128)], blocks, sem
        ).wait()
        iv = idx_flat[pl.ds(g * _L, _L)]
        col = lane * 128 + (iv & 127)
        vs = [
            plsc.load_gather(blocks, [jnp.full((_L,), j, jnp.int32), col])
            for j in range(ACTIONS)
        ]
        m = vs[0]
        for j in range(1, ACTIONS):
            m = jnp.maximum(m, vs[j])
        es = [jnp.exp(v - m) for v in vs]
        s = es[0]
        for j in range(1, ACTIONS):
            s = s + es[j]
        r = 1.0 / s
        opos = g * _L + lane
        for j in range(ACTIONS):
            plsc.store_scatter(
                outbuf, [jnp.full((_L,), j, jnp.int32), opos], es[j] * r
            )

    bufs = ((blk0, sem0), (blk1, sem1), (blk2, sem2))

    fire(0, blk0, sem0)
    fire(1, blk1, sem1)

    def triple(p, carry):
        g = p * 3
        for q in range(3):
            compute(g + q, *bufs[q])
            fire(g + q + 2, *bufs[(q + 2) % 3])
        return carry

    lax.fori_loop(0, _NGROUP // 3, triple, 0)
    compute(30, blk0, sem0)
    compute(31, blk1, sem1)

    pltpu.sync_copy(outbuf, out_hbm.at[:, pl.ds(base, _B_PER_W)])


@jax.jit
def _run(x, table):
    tab_t = table.T
    mesh = plsc.VectorSubcoreMesh(core_axis_name="c", subcore_axis_name="s")
    kern = functools.partial(
        pl.kernel,
        out_type=jax.ShapeDtypeStruct((ACTIONS, BATCH), jnp.float32),
        mesh=mesh,
        scratch_types=[
            pltpu.VMEM((_B_PER_W,), jnp.int32),
            pltpu.VMEM((ACTIONS, _L * 128), jnp.float32),
            pltpu.VMEM((ACTIONS, _L * 128), jnp.float32),
            pltpu.VMEM((ACTIONS, _L * 128), jnp.float32),
            pltpu.VMEM((ACTIONS, _B_PER_W), jnp.float32),
            pltpu.SemaphoreType.DMA,
            pltpu.SemaphoreType.DMA,
            pltpu.SemaphoreType.DMA,
        ],
        compiler_params=pltpu.CompilerParams(needs_layout_passes=False),
    )(_sc_body)
    out = kern(x.astype(jnp.int32), tab_t)
    return out.T


def kernel(x, table):
    return _run(x, table)
